# baseline (speedup 1.0000x reference)
import functools

import jax
import jax.numpy as jnp
from jax import lax
from jax.experimental import pallas as pl
from jax.experimental.pallas import tpu as pltpu
from jax.experimental.pallas import tpu_sc as plsc

N = 10000
E = 320000
D = 128
DH = D // 2

NC = 2
NS = 16
L = 16

SUPER = 256
K = 128
NSUP_T = 80
NPH = 4
SPP = NSUP_T // NPH
EPT = SUPER * NSUP_T
E_PAD = EPT * NS
NROW2D = E_PAD // SUPER
N_PAD = 10240
ROWS_PT = N_PAD // NS
DUMMY_ROW = 10100
RB = 64


def _mm_body(x_ref, wn_ref, h_ref):
    dn = (((1,), (1,)), ((), ()))
    h_ref[0] = lax.dot_general(x_ref[...], wn_ref[0], dn,
                               preferred_element_type=jnp.float32)


def _matmul_h(x, wn2):
    blk = 1000
    return pl.pallas_call(
        _mm_body,
        grid=(NC, N // blk),
        in_specs=[
            pl.BlockSpec((blk, D), lambda g, i: (i, 0)),
            pl.BlockSpec((1, DH, D), lambda g, i: (g, 0, 0)),
        ],
        out_specs=pl.BlockSpec((1, blk, DH), lambda g, i: (g, i, 0)),
        out_shape=jax.ShapeDtypeStruct((NC, N, DH), jnp.float32),
    )(x, wn2)


def _sc_body(h_hbm, src_hbm, dst_hbm, ps_hbm, pc_hbm,
             slab_s, slab_d, r0, r1, hist, zt, acc,
             gs0, gs1, ss0, ss1):
    c = lax.axis_index("c")
    s = lax.axis_index("s")
    rows = (r0, r1)
    gsem = (gs0, gs1)
    ssem = (ss0, ss1)
    row0_2d = s * NSUP_T
    arow0 = s * ROWS_PT

    def zt_row(r, carry):
        for j in range(DH // L):
            zt[r, pl.ds(j * L, L)] = jnp.zeros((L,), jnp.float32)
        return carry

    lax.fori_loop(0, RB, zt_row, 0)

    def hist_blk(i, carry):
        hist[pl.ds(i * L, L)] = jnp.zeros((L,), jnp.float32)
        return carry

    lax.fori_loop(0, N_PAD // L, hist_blk, 0)

    def zero_blk(j, carry):
        pltpu.sync_copy(zt, acc.at[pl.ds(arow0 + j * RB, RB), :])
        return carry

    lax.fori_loop(0, ROWS_PT // RB, zero_blk, 0)
    plsc.subcore_barrier()

    ones16 = jnp.ones((L,), jnp.float32)

    def run_phase(ph):
        base = ph * SPP
        pltpu.sync_copy(
            src_hbm.at[pl.ds(2 * (row0_2d + base), 2 * SPP), :], slab_s)
        pltpu.sync_copy(dst_hbm.at[c, pl.ds(row0_2d + base, SPP), :], slab_d)

        def hrow(r, carry):
            for j in range(K // L):
                idx16 = slab_s[r, pl.ds(j * L, L)]
                plsc.addupdate_scatter(hist, [idx16], ones16)
            return carry

        lax.fori_loop(0, 2 * SPP, hrow, 0)

        def g_issue(i, b):
            pltpu.async_copy(h_hbm.at[slab_d.at[i]], rows[b], gsem[b])

        def g_wait(i, b):
            pltpu.make_async_copy(
                h_hbm.at[slab_d.at[i]], rows[b], gsem[b]).wait()

        def s_issue(i, b):
            pltpu.async_copy(rows[b].at[pl.ds(0, K), :],
                             acc.at[slab_s.at[2 * i]], ssem[b], add=True)
            pltpu.async_copy(rows[b].at[pl.ds(K, K), :],
                             acc.at[slab_s.at[2 * i + 1]], ssem[b], add=True)

        def s_wait(i, b):
            pltpu.make_async_copy(rows[b].at[pl.ds(0, K), :],
                                  acc.at[slab_s.at[2 * i]], ssem[b]).wait()
            pltpu.make_async_copy(rows[b].at[pl.ds(K, K), :],
                                  acc.at[slab_s.at[2 * i + 1]],
                                  ssem[b]).wait()

        g_issue(0, 0)
        g_issue(1, 1)
        g_wait(0, 0)
        s_issue(0, 0)

        def main(k, carry):
            for off in range(2):
                i = 2 * k + off
                b = off
                s_wait(i - 2, b)
                g_issue(i, b)
                g_wait(i - 1, 1 - b)
                s_issue(i - 1, 1 - b)
            return carry

        lax.fori_loop(1, SPP // 2, main, 0)

        g_wait(SPP - 1, 1)
        s_issue(SPP - 1, 1)
        s_wait(SPP - 2, 0)
        s_wait(SPP - 1, 1)

    for ph in range(NPH):
        run_phase(ph)
    plsc.subcore_barrier()

    def out_blk(j, carry):
        r = arow0 + j * RB
        pltpu.sync_copy(acc.at[pl.ds(r, RB), :], zt)
        pltpu.sync_copy(zt, ps_hbm.at[c, pl.ds(r, RB), :])
        return carry

    lax.fori_loop(0, ROWS_PT // RB, out_blk, 0)
    pltpu.sync_copy(hist, pc_hbm.at[c, s])


_sc_call = functools.partial(
    pl.kernel,
    out_type=[
        jax.ShapeDtypeStruct((NC, N_PAD, DH), jnp.float32),
        jax.ShapeDtypeStruct((NC, NS, N_PAD), jnp.float32),
    ],
    mesh=plsc.VectorSubcoreMesh(core_axis_name="c", subcore_axis_name="s"),
    compiler_params=pltpu.CompilerParams(use_tc_tiling_on_sc=False,
                                        needs_layout_passes=False),
    scratch_types=[
        pltpu.VMEM((2 * SPP, K), jnp.int32),
        pltpu.VMEM((SPP, SUPER), jnp.int32),
        pltpu.VMEM((SUPER, DH), jnp.float32),
        pltpu.VMEM((SUPER, DH), jnp.float32),
        pltpu.VMEM((N_PAD,), jnp.float32),
        pltpu.VMEM((RB, DH), jnp.float32),
        pltpu.VMEM_SHARED((N_PAD, DH), jnp.float32),
        pltpu.SemaphoreType.DMA,
        pltpu.SemaphoreType.DMA,
        pltpu.SemaphoreType.DMA,
        pltpu.SemaphoreType.DMA,
    ],
)(_sc_body)


def _cnt_body(pc_ref, o_ref):
    o_ref[...] = jnp.sum(pc_ref[0], axis=0)[:, None]


def _cnt_reduce(pc):
    return pl.pallas_call(
        _cnt_body,
        out_shape=jax.ShapeDtypeStruct((N_PAD, 1), jnp.float32),
    )(pc)


def _fin_body(x_ref, ws_ref, ps_ref, pc_ref, o_ref):
    dn = (((1,), (1,)), ((), ()))
    sf = lax.dot_general(x_ref[...], ws_ref[...], dn,
                         preferred_element_type=jnp.float32)
    sums = jnp.concatenate([ps_ref[0], ps_ref[1]], axis=1)
    cnts = pc_ref[...]
    agg = sums / jnp.maximum(cnts, 1.0)
    neigh = jnp.where(agg >= 0, agg, 0.2 * agg)
    out = sf + neigh
    out = jnp.where(out >= 0, out, 0.2 * out)
    nrm = jnp.sqrt(jnp.sum(out * out, axis=1, keepdims=True))
    o_ref[...] = out / jnp.maximum(nrm, 1e-12)


def _finalize(x, ws, ps, pc):
    blk = 1000
    grid = N // blk
    return pl.pallas_call(
        _fin_body,
        grid=(grid,),
        in_specs=[
            pl.BlockSpec((blk, D), lambda i: (i, 0)),
            pl.BlockSpec((D, D), lambda i: (0, 0)),
            pl.BlockSpec((NC, blk, DH), lambda i: (0, i, 0)),
            pl.BlockSpec((blk, 1), lambda i: (i, 0)),
        ],
        out_specs=pl.BlockSpec((blk, D), lambda i: (i, 0)),
        out_shape=jax.ShapeDtypeStruct((N, D), jnp.float32),
    )(x, ws, ps, pc)


@jax.jit
def kernel(node_fts, edges, edge_fts, W_self, W_neigh):
    del edge_fts
    h = _matmul_h(node_fts, W_neigh.reshape(NC, DH, D))
    h_flat = h.reshape(NC * N, DH)
    pad = E_PAD - E
    src_p = jnp.concatenate(
        [edges[0], jnp.full((pad,), DUMMY_ROW, jnp.int32)])
    dst_p = jnp.concatenate([edges[1], jnp.zeros((pad,), jnp.int32)])
    src2d = src_p.reshape(2 * NROW2D, K)
    dst3d = jnp.stack([dst_p, dst_p + N]).reshape(NC, NROW2D, SUPER)
    ps, pc = _sc_call(h_flat, src2d, dst3d)
    cnt1 = _cnt_reduce(pc)
    return _finalize(node_fts, W_self, ps, cnt1)


# speedup vs baseline: 1.5414x; 1.5414x over previous
import functools

import jax
import jax.numpy as jnp
from jax import lax
from jax.experimental import pallas as pl
from jax.experimental.pallas import tpu as pltpu
from jax.experimental.pallas import tpu_sc as plsc

N = 10000
E = 320000
D = 128
DH = D // 2

NC = 2
NS = 16
L = 16

SUPER = 128
NSUP_T = 160
NPH = 4
SPP = NSUP_T // NPH
NBUF = 10
EPT = SUPER * NSUP_T
E_PAD = EPT * NS
NROW2D = E_PAD // SUPER
N_PAD = 10240
ROWS_PT = N_PAD // NS
DUMMY_ROW = 10100
RB = 64
ADT = jnp.bfloat16
AL = 32


def _mm_body(x_ref, wn_ref, h_ref):
    dn = (((1,), (1,)), ((), ()))
    h_ref[0] = lax.dot_general(
        x_ref[...], wn_ref[0], dn,
        preferred_element_type=jnp.float32).astype(ADT)


def _matmul_h(x, wn2):
    blk = 1000
    return pl.pallas_call(
        _mm_body,
        grid=(NC, N // blk),
        in_specs=[
            pl.BlockSpec((blk, D), lambda g, i: (i, 0)),
            pl.BlockSpec((1, DH, D), lambda g, i: (g, 0, 0)),
        ],
        out_specs=pl.BlockSpec((1, blk, DH), lambda g, i: (g, i, 0)),
        out_shape=jax.ShapeDtypeStruct((NC, N, DH), ADT),
    )(x, wn2)


def _sc_body(h_hbm, src_hbm, dst_hbm, ps_hbm, pc_hbm,
             slab_s, slab_d,
             r0, r1, r2, r3, r4, r5, r6, r7, r8, r9, hist, zt, acc,
             gs0, gs1, gs2, gs3, gs4, gs5, gs6, gs7, gs8, gs9,
             ss0, ss1, ss2, ss3, ss4, ss5, ss6, ss7, ss8, ss9):
    c = lax.axis_index("c")
    s = lax.axis_index("s")
    rows = (r0, r1, r2, r3, r4, r5, r6, r7, r8, r9)
    gsem = (gs0, gs1, gs2, gs3, gs4, gs5, gs6, gs7, gs8, gs9)
    ssem = (ss0, ss1, ss2, ss3, ss4, ss5, ss6, ss7, ss8, ss9)
    row0_2d = s * NSUP_T
    arow0 = s * ROWS_PT

    def zt_row(r, carry):
        for j in range(DH // AL):
            zt[r, pl.ds(j * AL, AL)] = jnp.zeros((AL,), ADT)
        return carry

    lax.fori_loop(0, RB, zt_row, 0)

    def hist_blk(i, carry):
        hist[pl.ds(i * L, L)] = jnp.zeros((L,), jnp.float32)
        return carry

    lax.fori_loop(0, N_PAD // L, hist_blk, 0)

    def zero_blk(j, carry):
        pltpu.sync_copy(zt, acc.at[pl.ds(arow0 + j * RB, RB), :])
        return carry

    lax.fori_loop(0, ROWS_PT // RB, zero_blk, 0)
    plsc.subcore_barrier()

    ones16 = jnp.ones((L,), jnp.float32)

    def run_phase(ph):
        base = ph * SPP
        pltpu.sync_copy(src_hbm.at[pl.ds(row0_2d + base, SPP), :], slab_s)
        pltpu.sync_copy(dst_hbm.at[c, pl.ds(row0_2d + base, SPP), :], slab_d)

        @pl.when(c == 0)
        def _hist():
            def hrow(r, carry):
                for j in range(SUPER // L):
                    idx16 = slab_s[r, pl.ds(j * L, L)]
                    plsc.addupdate_scatter(hist, [idx16], ones16)
                return carry

            lax.fori_loop(0, SPP, hrow, 0)

        def g_issue(i, b):
            pltpu.async_copy(h_hbm.at[slab_d.at[i]], rows[b], gsem[b])

        def g_wait(i, b):
            pltpu.make_async_copy(
                h_hbm.at[slab_d.at[i]], rows[b], gsem[b]).wait()

        def s_issue(i, b):
            pltpu.async_copy(rows[b], acc.at[slab_s.at[i]], ssem[b], add=True)

        def s_wait(i, b):
            pltpu.make_async_copy(
                rows[b], acc.at[slab_s.at[i]], ssem[b]).wait()

        half = NBUF // 2
        for i in range(half):
            g_issue(i, i)
        for i in range(half, NBUF):
            g_issue(i, i)
            g_wait(i - half, i - half)
            s_issue(i - half, i - half)

        def main(k, carry):
            for off in range(NBUF):
                i = NBUF * k + off
                s_wait(i - NBUF, off)
                g_issue(i, off)
                j = i - half
                bj = (off + half) % NBUF
                g_wait(j, bj)
                s_issue(j, bj)
            return carry

        lax.fori_loop(1, SPP // NBUF, main, 0)

        for j in range(SPP - half, SPP):
            g_wait(j, j % NBUF)
            s_issue(j, j % NBUF)
        for j in range(SPP - NBUF, SPP):
            s_wait(j, j % NBUF)

    for ph in range(NPH):
        run_phase(ph)
    plsc.subcore_barrier()

    def out_blk(j, carry):
        r = arow0 + j * RB
        pltpu.sync_copy(acc.at[pl.ds(r, RB), :], zt)
        pltpu.sync_copy(zt, ps_hbm.at[c, pl.ds(r, RB), :])
        return carry

    lax.fori_loop(0, ROWS_PT // RB, out_blk, 0)

    @pl.when(c == 0)
    def _hist_out():
        pltpu.sync_copy(hist, pc_hbm.at[s])


_sc_call = functools.partial(
    pl.kernel,
    out_type=[
        jax.ShapeDtypeStruct((NC, N_PAD, DH), ADT),
        jax.ShapeDtypeStruct((NS, N_PAD), jnp.float32),
    ],
    mesh=plsc.VectorSubcoreMesh(core_axis_name="c", subcore_axis_name="s"),
    compiler_params=pltpu.CompilerParams(use_tc_tiling_on_sc=False,
                                        needs_layout_passes=False),
    scratch_types=[
        pltpu.VMEM((SPP, SUPER), jnp.int32),
        pltpu.VMEM((SPP, SUPER), jnp.int32),
        pltpu.VMEM((SUPER, DH), ADT),
        pltpu.VMEM((SUPER, DH), ADT),
        pltpu.VMEM((SUPER, DH), ADT),
        pltpu.VMEM((SUPER, DH), ADT),
        pltpu.VMEM((SUPER, DH), ADT),
        pltpu.VMEM((SUPER, DH), ADT),
        pltpu.VMEM((SUPER, DH), ADT),
        pltpu.VMEM((SUPER, DH), ADT),
        pltpu.VMEM((SUPER, DH), ADT),
        pltpu.VMEM((SUPER, DH), ADT),
        pltpu.VMEM((N_PAD,), jnp.float32),
        pltpu.VMEM((RB, DH), ADT),
        pltpu.VMEM_SHARED((N_PAD, DH), ADT),
        pltpu.SemaphoreType.DMA,
        pltpu.SemaphoreType.DMA,
        pltpu.SemaphoreType.DMA,
        pltpu.SemaphoreType.DMA,
        pltpu.SemaphoreType.DMA,
        pltpu.SemaphoreType.DMA,
        pltpu.SemaphoreType.DMA,
        pltpu.SemaphoreType.DMA,
        pltpu.SemaphoreType.DMA,
        pltpu.SemaphoreType.DMA,
        pltpu.SemaphoreType.DMA,
        pltpu.SemaphoreType.DMA,
        pltpu.SemaphoreType.DMA,
        pltpu.SemaphoreType.DMA,
        pltpu.SemaphoreType.DMA,
        pltpu.SemaphoreType.DMA,
        pltpu.SemaphoreType.DMA,
        pltpu.SemaphoreType.DMA,
        pltpu.SemaphoreType.DMA,
        pltpu.SemaphoreType.DMA,
    ],
)(_sc_body)


def _cnt_body(pc_ref, o_ref):
    o_ref[...] = jnp.sum(pc_ref[...], axis=0)[:, None]


def _cnt_reduce(pc):
    return pl.pallas_call(
        _cnt_body,
        out_shape=jax.ShapeDtypeStruct((N_PAD, 1), jnp.float32),
    )(pc)


def _fin_body(x_ref, ws_ref, ps_ref, pc_ref, o_ref):
    dn = (((1,), (1,)), ((), ()))
    sf = lax.dot_general(x_ref[...], ws_ref[...], dn,
                         preferred_element_type=jnp.float32)
    sums = jnp.concatenate(
        [ps_ref[0], ps_ref[1]], axis=1).astype(jnp.float32)
    cnts = pc_ref[...]
    agg = sums / jnp.maximum(cnts, 1.0)
    neigh = jnp.where(agg >= 0, agg, 0.2 * agg)
    out = sf + neigh
    out = jnp.where(out >= 0, out, 0.2 * out)
    nrm = jnp.sqrt(jnp.sum(out * out, axis=1, keepdims=True))
    o_ref[...] = out / jnp.maximum(nrm, 1e-12)


def _finalize(x, ws, ps, pc):
    blk = 1000
    grid = N // blk
    return pl.pallas_call(
        _fin_body,
        grid=(grid,),
        in_specs=[
            pl.BlockSpec((blk, D), lambda i: (i, 0)),
            pl.BlockSpec((D, D), lambda i: (0, 0)),
            pl.BlockSpec((NC, blk, DH), lambda i: (0, i, 0)),
            pl.BlockSpec((blk, 1), lambda i: (i, 0)),
        ],
        out_specs=pl.BlockSpec((blk, D), lambda i: (i, 0)),
        out_shape=jax.ShapeDtypeStruct((N, D), jnp.float32),
    )(x, ws, ps, pc)


@jax.jit
def kernel(node_fts, edges, edge_fts, W_self, W_neigh):
    del edge_fts
    h = _matmul_h(node_fts, W_neigh.reshape(NC, DH, D))
    h_flat = h.reshape(NC * N, DH)
    pad = E_PAD - E
    src_p = jnp.concatenate(
        [edges[0], jnp.full((pad,), DUMMY_ROW, jnp.int32)])
    dst_p = jnp.concatenate([edges[1], jnp.zeros((pad,), jnp.int32)])
    src2d = src_p.reshape(NROW2D, SUPER)
    dst3d = jnp.stack([dst_p, dst_p + N]).reshape(NC, NROW2D, SUPER)
    ps, pc = _sc_call(h_flat, src2d, dst3d)
    cnt1 = _cnt_reduce(pc)
    return _finalize(node_fts, W_self, ps, cnt1)
